# baseline (speedup 1.0000x reference)
import functools

import jax
import jax.numpy as jnp
from jax import lax
from jax.experimental import pallas as pl
from jax.experimental.pallas import tpu as pltpu
from jax.experimental.pallas import tpu_sc as plsc

N = 10000
E = 320000
CH = 128
OUT = 64
NSEED = 1024

NC = 2
NS = 16
NW = NC * NS

N_PAD = 12288
RPW = N_PAD // NW
M_PAD = 10240
MRPT = M_PAD // NS
E_PAD = 327680
EPW = E_PAD // NW
CHUNK = 128
NCHUNK = EPW // CHUNK

BM = 256
NBLK = M_PAD // BM

_sc_mesh = plsc.VectorSubcoreMesh(
    core_axis_name="c", subcore_axis_name="s", num_cores=NC, num_subcores=NS)


@functools.partial(
    pl.kernel,
    mesh=_sc_mesh,
    out_type=[
        jax.ShapeDtypeStruct((N_PAD, CH), jnp.float32),
        jax.ShapeDtypeStruct((N_PAD, 16), jnp.float32),
    ],
    scratch_types=[
        pltpu.VMEM((RPW // CHUNK, CHUNK), jnp.int32),
        pltpu.VMEM((RPW // CHUNK, CHUNK), jnp.int32),
        [pltpu.VMEM((CHUNK, CH), jnp.float32)] * 3,
        [pltpu.VMEM((CHUNK, 16), jnp.float32)] * 3,
        pltpu.SemaphoreType.DMA,
        pltpu.SemaphoreType.DMA,
    ],
    compiler_params=pltpu.CompilerParams(use_tc_tiling_on_sc=False),
)
def _gather_sc(emb_hbm, nid3_hbm, seed16_hbm, bid3_hbm,
               emb_out, st_out,
               nid_all, bid_all, ebufs, sbufs, gsem, osem):
    c = lax.axis_index("c")
    s = lax.axis_index("s")
    wid = c * NS + s
    base = wid * RPW
    nck = RPW // CHUNK

    pltpu.sync_copy(nid3_hbm.at[wid], nid_all)
    pltpu.sync_copy(bid3_hbm.at[wid], bid_all)

    for k in range(nck):
        pltpu.async_copy(emb_hbm.at[nid_all.at[k]], ebufs[k], gsem)
        pltpu.async_copy(seed16_hbm.at[bid_all.at[k]], sbufs[k], gsem)
    for k in range(nck):
        pltpu.make_async_copy(emb_hbm.at[nid_all.at[k]], ebufs[k], gsem).wait()
        pltpu.async_copy(ebufs[k], emb_out.at[pl.ds(base + k * CHUNK, CHUNK)], osem)
        pltpu.make_async_copy(seed16_hbm.at[bid_all.at[k]], sbufs[k], gsem).wait()
        pltpu.async_copy(sbufs[k], st_out.at[pl.ds(base + k * CHUNK, CHUNK)], osem)
    for k in range(nck):
        pltpu.make_async_copy(ebufs[k], emb_out.at[pl.ds(base + k * CHUNK, CHUNK)],
                              osem).wait()
        pltpu.make_async_copy(sbufs[k], st_out.at[pl.ds(base + k * CHUNK, CHUNK)],
                              osem).wait()


@functools.partial(
    pl.kernel,
    mesh=_sc_mesh,
    out_type=[
        jax.ShapeDtypeStruct((NC * M_PAD, CH), jnp.float32),
        jax.ShapeDtypeStruct((NC * M_PAD, 16), jnp.float32),
    ],
    scratch_types=[
        pltpu.VMEM((CHUNK,), jnp.int32),
        pltpu.VMEM((CHUNK,), jnp.int32),
        pltpu.VMEM((CHUNK,), jnp.int32),
        pltpu.VMEM((CHUNK,), jnp.int32),
        pltpu.VMEM((CHUNK, CH), jnp.float32),
        pltpu.VMEM((CHUNK, CH), jnp.float32),
        pltpu.VMEM((CHUNK, 16), jnp.float32),
        pltpu.VMEM_SHARED((M_PAD, CH), jnp.float32),
        pltpu.VMEM_SHARED((M_PAD, 16), jnp.float32),
        pltpu.SemaphoreType.DMA,
        pltpu.SemaphoreType.DMA,
        pltpu.SemaphoreType.DMA,
    ],
    compiler_params=pltpu.CompilerParams(use_tc_tiling_on_sc=False),
)
def _edge_agg_sc(h_hbm, src_hbm, dst_hbm, zagg_hbm, zdeg_hbm, ones_hbm,
                 agg_out, deg_out,
                 sidx0, sidx1, didx0, didx1, rows0, rows1, ones_v,
                 agg_sh, deg_sh, gsem, ssem, isem):
    c = lax.axis_index("c")
    s = lax.axis_index("s")
    wid = c * NS + s
    ebase = wid * EPW
    pltpu.sync_copy(zagg_hbm.at[pl.ds(s * MRPT, MRPT)],
                    agg_sh.at[pl.ds(s * MRPT, MRPT)])
    pltpu.sync_copy(zdeg_hbm.at[pl.ds(s * MRPT, MRPT)],
                    deg_sh.at[pl.ds(s * MRPT, MRPT)])
    pltpu.sync_copy(ones_hbm, ones_v)
    pltpu.sync_copy(src_hbm.at[pl.ds(ebase, CHUNK)], sidx0)
    pltpu.sync_copy(src_hbm.at[pl.ds(ebase + CHUNK, CHUNK)], sidx1)
    pltpu.async_copy(dst_hbm.at[pl.ds(ebase, CHUNK)], didx0, isem)
    pltpu.async_copy(dst_hbm.at[pl.ds(ebase + CHUNK, CHUNK)], didx1, isem)
    pltpu.async_copy(h_hbm.at[sidx0], rows0, gsem)
    pltpu.async_copy(h_hbm.at[sidx1], rows1, gsem)
    plsc.subcore_barrier()

    @pl.loop(0, NCHUNK, step=2)
    def _(j):
        pltpu.make_async_copy(h_hbm.at[sidx0], rows0, gsem).wait()
        pltpu.make_async_copy(dst_hbm.at[pl.ds(ebase, CHUNK)], didx0, isem).wait()
        pltpu.async_copy(src_hbm.at[pl.ds(ebase + (j + 2) * CHUNK, CHUNK)],
                         sidx0, isem)
        pltpu.async_copy(rows0, agg_sh.at[didx0], ssem, add=True)
        pltpu.async_copy(ones_v, deg_sh.at[didx0], ssem, add=True)

        pltpu.make_async_copy(h_hbm.at[sidx1], rows1, gsem).wait()
        pltpu.make_async_copy(dst_hbm.at[pl.ds(ebase, CHUNK)], didx1, isem).wait()
        pltpu.async_copy(src_hbm.at[pl.ds(ebase + (j + 3) * CHUNK, CHUNK)],
                         sidx1, isem)
        pltpu.async_copy(rows1, agg_sh.at[didx1], ssem, add=True)
        pltpu.async_copy(ones_v, deg_sh.at[didx1], ssem, add=True)

        pltpu.make_async_copy(rows0, agg_sh.at[didx0], ssem).wait()
        pltpu.make_async_copy(ones_v, deg_sh.at[didx0], ssem).wait()
        pltpu.async_copy(dst_hbm.at[pl.ds(ebase + (j + 2) * CHUNK, CHUNK)],
                         didx0, isem)
        pltpu.make_async_copy(src_hbm.at[pl.ds(ebase, CHUNK)], sidx0, isem).wait()
        pltpu.async_copy(h_hbm.at[sidx0], rows0, gsem)

        pltpu.make_async_copy(rows1, agg_sh.at[didx1], ssem).wait()
        pltpu.make_async_copy(ones_v, deg_sh.at[didx1], ssem).wait()
        pltpu.async_copy(dst_hbm.at[pl.ds(ebase + (j + 3) * CHUNK, CHUNK)],
                         didx1, isem)
        pltpu.make_async_copy(src_hbm.at[pl.ds(ebase, CHUNK)], sidx1, isem).wait()
        pltpu.async_copy(h_hbm.at[sidx1], rows1, gsem)

    pltpu.make_async_copy(h_hbm.at[sidx0], rows0, gsem).wait()
    pltpu.make_async_copy(h_hbm.at[sidx1], rows1, gsem).wait()
    pltpu.make_async_copy(dst_hbm.at[pl.ds(ebase, CHUNK)], didx0, isem).wait()
    pltpu.make_async_copy(dst_hbm.at[pl.ds(ebase, CHUNK)], didx1, isem).wait()
    plsc.subcore_barrier()
    pltpu.sync_copy(agg_sh.at[pl.ds(s * MRPT, MRPT)],
                    agg_out.at[pl.ds(c * M_PAD + s * MRPT, MRPT)])
    pltpu.sync_copy(deg_sh.at[pl.ds(s * MRPT, MRPT)],
                    deg_out.at[pl.ds(c * M_PAD + s * MRPT, MRPT)])


def _encode_tc_body(x_ref, st_ref, nt_ref, emb_ref, we_ref, be_ref, wt_ref, bt_ref,
                    h_ref):
    h = jnp.tanh(jnp.dot(x_ref[...], we_ref[...],
                         preferred_element_type=jnp.float32) + be_ref[...])
    rel = st_ref[...] - nt_ref[...]
    h_ref[...] = h + rel * wt_ref[...] + bt_ref[...] + emb_ref[...]


_encode_tc = pl.pallas_call(
    _encode_tc_body,
    grid=(N_PAD // BM,),
    in_specs=[
        pl.BlockSpec((BM, CH), lambda i: (i, 0)),
        pl.BlockSpec((BM, 1), lambda i: (i, 0)),
        pl.BlockSpec((BM, 1), lambda i: (i, 0)),
        pl.BlockSpec((BM, CH), lambda i: (i, 0)),
        pl.BlockSpec((CH, CH), lambda i: (0, 0)),
        pl.BlockSpec((1, CH), lambda i: (0, 0)),
        pl.BlockSpec((1, CH), lambda i: (0, 0)),
        pl.BlockSpec((1, CH), lambda i: (0, 0)),
    ],
    out_specs=pl.BlockSpec((BM, CH), lambda i: (i, 0)),
    out_shape=jax.ShapeDtypeStruct((N_PAD, CH), jnp.float32),
)


def _head_tc_body(h_ref, agg_ref, deg_ref, ws_ref, wn_ref, wh_ref, bh_ref, out_ref):
    a = agg_ref[0] + agg_ref[1]
    d = deg_ref[0][:, 0:1] + deg_ref[1][:, 0:1]
    a = a / jnp.maximum(d, 1.0)
    h = h_ref[...]
    h2 = jnp.maximum(
        jnp.dot(h, ws_ref[...], preferred_element_type=jnp.float32)
        + jnp.dot(a, wn_ref[...], preferred_element_type=jnp.float32), 0.0)
    out_ref[...] = jnp.dot(h2, wh_ref[...],
                           preferred_element_type=jnp.float32) + bh_ref[...]


_head_tc = pl.pallas_call(
    _head_tc_body,
    grid=(NBLK,),
    in_specs=[
        pl.BlockSpec((BM, CH), lambda i: (i, 0)),
        pl.BlockSpec((NC, BM, CH), lambda i: (0, i, 0)),
        pl.BlockSpec((NC, BM, 16), lambda i: (0, i, 0)),
        pl.BlockSpec((CH, CH), lambda i: (0, 0)),
        pl.BlockSpec((CH, CH), lambda i: (0, 0)),
        pl.BlockSpec((CH, OUT), lambda i: (0, 0)),
        pl.BlockSpec((1, OUT), lambda i: (0, 0)),
    ],
    out_specs=pl.BlockSpec((BM, OUT), lambda i: (i, 0)),
    out_shape=jax.ShapeDtypeStruct((M_PAD, OUT), jnp.float32),
)


def kernel(x, node_time, seed_time, W_enc, b_enc, W_time, b_time, emb_table,
           W_self, W_nbr, W_head, b_head, edge_index, batch_ids, n_id):
    f32 = jnp.float32
    pad_n = N_PAD - N
    pad_e = E_PAD - E

    x_p = jnp.pad(x.astype(f32), ((0, pad_n), (0, 0)))
    nt_p = jnp.pad(node_time.astype(f32), (0, pad_n)).reshape(N_PAD, 1)
    n_id_p = jnp.pad(n_id.astype(jnp.int32), (0, pad_n))
    bid_p = jnp.pad(batch_ids.astype(jnp.int32), (0, pad_n))
    src_p = jnp.pad(edge_index[0].astype(jnp.int32), (0, pad_e + 2 * CHUNK))
    dst_p = jnp.pad(edge_index[1].astype(jnp.int32), (0, pad_e),
                    constant_values=M_PAD - 1)

    zdeg = jnp.zeros((M_PAD, 16), f32)
    ones_rows = jnp.ones((CHUNK, 16), f32)
    zagg = jnp.zeros((M_PAD, CH), f32)
    seed16 = jnp.broadcast_to(seed_time.astype(f32)[:, None], (NSEED, 16))

    nid3 = n_id_p.reshape(NW, RPW // CHUNK, CHUNK)
    bid3 = bid_p.reshape(NW, RPW // CHUNK, CHUNK)
    dst_pp = jnp.pad(dst_p, (0, 2 * CHUNK))

    emb_rows, st16 = _gather_sc(emb_table.astype(f32), nid3, seed16, bid3)
    h = _encode_tc(x_p, st16[:, 0:1], nt_p, emb_rows,
                   W_enc.astype(f32), b_enc.astype(f32).reshape(1, CH),
                   W_time.astype(f32), b_time.astype(f32).reshape(1, CH))
    agg2, deg2 = _edge_agg_sc(h, src_p, dst_pp, zagg, zdeg, ones_rows)
    out = _head_tc(h, agg2.reshape(NC, M_PAD, CH), deg2.reshape(NC, M_PAD, 16),
                   W_self.astype(f32), W_nbr.astype(f32), W_head.astype(f32),
                   b_head.astype(f32).reshape(1, OUT))
    return out[:N]


# speedup vs baseline: 1.2884x; 1.2884x over previous
import functools

import jax
import jax.numpy as jnp
from jax import lax
from jax.experimental import pallas as pl
from jax.experimental.pallas import tpu as pltpu
from jax.experimental.pallas import tpu_sc as plsc

N = 10000
E = 320000
CH = 128
OUT = 64
NSEED = 1024

NC = 2
NS = 16
NW = NC * NS

N_PAD = 12288
RPW = N_PAD // NW
M_PAD = 10240
MRPT = M_PAD // NS
E_PAD = 327680
EPW = E_PAD // NW
CHUNK = 128
NCHUNK = EPW // CHUNK

BM = 256
NBLK = M_PAD // BM

_sc_mesh = plsc.VectorSubcoreMesh(
    core_axis_name="c", subcore_axis_name="s", num_cores=NC, num_subcores=NS)


@functools.partial(
    pl.kernel,
    mesh=_sc_mesh,
    out_type=[
        jax.ShapeDtypeStruct((N_PAD, CH), jnp.float32),
        jax.ShapeDtypeStruct((N_PAD, 16), jnp.float32),
        jax.ShapeDtypeStruct((NC * M_PAD, 16), jnp.float32),
    ],
    scratch_types=[
        pltpu.VMEM((RPW // CHUNK, CHUNK), jnp.int32),
        pltpu.VMEM((RPW // CHUNK, CHUNK), jnp.int32),
        pltpu.VMEM((NCHUNK, CHUNK), jnp.int32),
        [pltpu.VMEM((CHUNK, CH), jnp.float32)] * 3,
        [pltpu.VMEM((CHUNK, 16), jnp.float32)] * 3,
        pltpu.VMEM((CHUNK, 16), jnp.float32),
        pltpu.VMEM_SHARED((M_PAD, 16), jnp.float32),
        pltpu.SemaphoreType.DMA,
        pltpu.SemaphoreType.DMA,
        pltpu.SemaphoreType.DMA,
    ],
    compiler_params=pltpu.CompilerParams(use_tc_tiling_on_sc=False),
)
def _gather_deg_sc(emb_hbm, nid3_hbm, seed16_hbm, bid3_hbm, dst3_hbm, zdeg_hbm,
                   ones_hbm,
                   emb_out, st_out, deg_out,
                   nid_all, bid_all, didx_all, ebufs, sbufs, ones_v, deg_sh,
                   gsem, ssem, osem):
    c = lax.axis_index("c")
    s = lax.axis_index("s")
    wid = c * NS + s
    base = wid * RPW
    nck = RPW // CHUNK

    pltpu.sync_copy(nid3_hbm.at[wid], nid_all)
    pltpu.sync_copy(bid3_hbm.at[wid], bid_all)
    pltpu.sync_copy(dst3_hbm.at[wid], didx_all)
    pltpu.sync_copy(ones_hbm, ones_v)
    pltpu.sync_copy(zdeg_hbm.at[pl.ds(s * MRPT, MRPT)],
                    deg_sh.at[pl.ds(s * MRPT, MRPT)])
    plsc.subcore_barrier()

    for k in range(nck):
        pltpu.async_copy(emb_hbm.at[nid_all.at[k]], ebufs[k], gsem)
        pltpu.async_copy(seed16_hbm.at[bid_all.at[k]], sbufs[k], gsem)
    for k in range(nck):
        pltpu.make_async_copy(emb_hbm.at[nid_all.at[k]], ebufs[k], gsem).wait()
        pltpu.async_copy(ebufs[k], emb_out.at[pl.ds(base + k * CHUNK, CHUNK)], osem)
        pltpu.make_async_copy(seed16_hbm.at[bid_all.at[k]], sbufs[k], gsem).wait()
        pltpu.async_copy(sbufs[k], st_out.at[pl.ds(base + k * CHUNK, CHUNK)], osem)

    @pl.loop(0, NCHUNK, step=8)
    def _(j):
        for t in range(8):
            pltpu.async_copy(ones_v, deg_sh.at[didx_all.at[j + t]], ssem, add=True)
        for t in range(8):
            pltpu.make_async_copy(ones_v, deg_sh.at[didx_all.at[j + t]], ssem).wait()

    for k in range(nck):
        pltpu.make_async_copy(ebufs[k], emb_out.at[pl.ds(base + k * CHUNK, CHUNK)],
                              osem).wait()
        pltpu.make_async_copy(sbufs[k], st_out.at[pl.ds(base + k * CHUNK, CHUNK)],
                              osem).wait()
    plsc.subcore_barrier()
    pltpu.sync_copy(deg_sh.at[pl.ds(s * MRPT, MRPT)],
                    deg_out.at[pl.ds(c * M_PAD + s * MRPT, MRPT)])


@functools.partial(
    pl.kernel,
    mesh=_sc_mesh,
    out_type=jax.ShapeDtypeStruct((NC * M_PAD, CH), jnp.float32),
    scratch_types=[
        pltpu.VMEM((CHUNK,), jnp.int32),
        pltpu.VMEM((CHUNK,), jnp.int32),
        pltpu.VMEM((NCHUNK, CHUNK), jnp.int32),
        pltpu.VMEM((CHUNK, CH), jnp.float32),
        pltpu.VMEM((CHUNK, CH), jnp.float32),
        pltpu.VMEM_SHARED((M_PAD, CH), jnp.float32),
        pltpu.SemaphoreType.DMA,
        pltpu.SemaphoreType.DMA,
        pltpu.SemaphoreType.DMA,
    ],
)
def _edge_agg_sc(h_hbm, src_hbm, dst3_hbm, zeros_hbm, agg_out,
                 sidx0, sidx1, didx_all, rows0, rows1, agg_sh,
                 gsem, ssem, isem):
    c = lax.axis_index("c")
    s = lax.axis_index("s")
    wid = c * NS + s
    ebase = wid * EPW
    pltpu.sync_copy(zeros_hbm.at[pl.ds(s * MRPT, MRPT)],
                    agg_sh.at[pl.ds(s * MRPT, MRPT)])
    pltpu.sync_copy(dst3_hbm.at[wid], didx_all)
    pltpu.sync_copy(src_hbm.at[pl.ds(ebase, CHUNK)], sidx0)
    pltpu.sync_copy(src_hbm.at[pl.ds(ebase + CHUNK, CHUNK)], sidx1)
    pltpu.async_copy(h_hbm.at[sidx0], rows0, gsem)
    pltpu.async_copy(h_hbm.at[sidx1], rows1, gsem)
    plsc.subcore_barrier()

    @pl.loop(0, NCHUNK, step=2)
    def _(j):
        pltpu.make_async_copy(h_hbm.at[sidx0], rows0, gsem).wait()
        pltpu.async_copy(src_hbm.at[pl.ds(ebase + (j + 2) * CHUNK, CHUNK)],
                         sidx0, isem)
        pltpu.async_copy(rows0, agg_sh.at[didx_all.at[j]], ssem, add=True)
        pltpu.make_async_copy(h_hbm.at[sidx1], rows1, gsem).wait()
        pltpu.async_copy(src_hbm.at[pl.ds(ebase + (j + 3) * CHUNK, CHUNK)],
                         sidx1, isem)
        pltpu.async_copy(rows1, agg_sh.at[didx_all.at[j + 1]], ssem, add=True)

        pltpu.make_async_copy(src_hbm.at[pl.ds(ebase, CHUNK)], sidx0, isem).wait()
        pltpu.make_async_copy(rows0, agg_sh.at[didx_all.at[j]], ssem).wait()
        pltpu.async_copy(h_hbm.at[sidx0], rows0, gsem)
        pltpu.make_async_copy(src_hbm.at[pl.ds(ebase, CHUNK)], sidx1, isem).wait()
        pltpu.make_async_copy(rows1, agg_sh.at[didx_all.at[j + 1]], ssem).wait()
        pltpu.async_copy(h_hbm.at[sidx1], rows1, gsem)

    pltpu.make_async_copy(h_hbm.at[sidx0], rows0, gsem).wait()
    pltpu.make_async_copy(h_hbm.at[sidx1], rows1, gsem).wait()
    plsc.subcore_barrier()
    pltpu.sync_copy(agg_sh.at[pl.ds(s * MRPT, MRPT)],
                    agg_out.at[pl.ds(c * M_PAD + s * MRPT, MRPT)])


def _encode_tc_body(x_ref, st_ref, nt_ref, emb_ref, we_ref, be_ref, wt_ref, bt_ref,
                    h_ref):
    h = jnp.tanh(jnp.dot(x_ref[...], we_ref[...],
                         preferred_element_type=jnp.float32) + be_ref[...])
    rel = st_ref[...] - nt_ref[...]
    h_ref[...] = h + rel * wt_ref[...] + bt_ref[...] + emb_ref[...]


_encode_tc = pl.pallas_call(
    _encode_tc_body,
    grid=(N_PAD // BM,),
    in_specs=[
        pl.BlockSpec((BM, CH), lambda i: (i, 0)),
        pl.BlockSpec((BM, 1), lambda i: (i, 0)),
        pl.BlockSpec((BM, 1), lambda i: (i, 0)),
        pl.BlockSpec((BM, CH), lambda i: (i, 0)),
        pl.BlockSpec((CH, CH), lambda i: (0, 0)),
        pl.BlockSpec((1, CH), lambda i: (0, 0)),
        pl.BlockSpec((1, CH), lambda i: (0, 0)),
        pl.BlockSpec((1, CH), lambda i: (0, 0)),
    ],
    out_specs=pl.BlockSpec((BM, CH), lambda i: (i, 0)),
    out_shape=jax.ShapeDtypeStruct((N_PAD, CH), jnp.float32),
)


def _head_tc_body(h_ref, agg_ref, deg_ref, ws_ref, wn_ref, wh_ref, bh_ref, out_ref):
    a = agg_ref[0] + agg_ref[1]
    d = deg_ref[0][:, 0:1] + deg_ref[1][:, 0:1]
    a = a / jnp.maximum(d, 1.0)
    h = h_ref[...]
    h2 = jnp.maximum(
        jnp.dot(h, ws_ref[...], preferred_element_type=jnp.float32)
        + jnp.dot(a, wn_ref[...], preferred_element_type=jnp.float32), 0.0)
    out_ref[...] = jnp.dot(h2, wh_ref[...],
                           preferred_element_type=jnp.float32) + bh_ref[...]


_head_tc = pl.pallas_call(
    _head_tc_body,
    grid=(NBLK,),
    in_specs=[
        pl.BlockSpec((BM, CH), lambda i: (i, 0)),
        pl.BlockSpec((NC, BM, CH), lambda i: (0, i, 0)),
        pl.BlockSpec((NC, BM, 16), lambda i: (0, i, 0)),
        pl.BlockSpec((CH, CH), lambda i: (0, 0)),
        pl.BlockSpec((CH, CH), lambda i: (0, 0)),
        pl.BlockSpec((CH, OUT), lambda i: (0, 0)),
        pl.BlockSpec((1, OUT), lambda i: (0, 0)),
    ],
    out_specs=pl.BlockSpec((BM, OUT), lambda i: (i, 0)),
    out_shape=jax.ShapeDtypeStruct((M_PAD, OUT), jnp.float32),
)


def kernel(x, node_time, seed_time, W_enc, b_enc, W_time, b_time, emb_table,
           W_self, W_nbr, W_head, b_head, edge_index, batch_ids, n_id):
    f32 = jnp.float32
    pad_n = N_PAD - N
    pad_e = E_PAD - E

    x_p = jnp.pad(x.astype(f32), ((0, pad_n), (0, 0)))
    nt_p = jnp.pad(node_time.astype(f32), (0, pad_n)).reshape(N_PAD, 1)
    n_id_p = jnp.pad(n_id.astype(jnp.int32), (0, pad_n))
    bid_p = jnp.pad(batch_ids.astype(jnp.int32), (0, pad_n))
    src_p = jnp.pad(edge_index[0].astype(jnp.int32), (0, pad_e + 2 * CHUNK))
    dst_p = jnp.pad(edge_index[1].astype(jnp.int32), (0, pad_e),
                    constant_values=M_PAD - 1)

    zdeg = jnp.zeros((M_PAD, 16), f32)
    ones_rows = jnp.ones((CHUNK, 16), f32)
    zagg = jnp.zeros((M_PAD, CH), f32)
    seed16 = jnp.broadcast_to(seed_time.astype(f32)[:, None], (NSEED, 16))

    nid3 = n_id_p.reshape(NW, RPW // CHUNK, CHUNK)
    bid3 = bid_p.reshape(NW, RPW // CHUNK, CHUNK)
    dst3 = dst_p.reshape(NW, NCHUNK, CHUNK)

    emb_rows, st16, deg2 = _gather_deg_sc(
        emb_table.astype(f32), nid3, seed16, bid3, dst3,
        zdeg, ones_rows)
    h = _encode_tc(x_p, st16[:, 0:1], nt_p, emb_rows,
                   W_enc.astype(f32), b_enc.astype(f32).reshape(1, CH),
                   W_time.astype(f32), b_time.astype(f32).reshape(1, CH))
    agg2 = _edge_agg_sc(h, src_p, dst3, zagg)
    out = _head_tc(h, agg2.reshape(NC, M_PAD, CH), deg2.reshape(NC, M_PAD, 16),
                   W_self.astype(f32), W_nbr.astype(f32), W_head.astype(f32),
                   b_head.astype(f32).reshape(1, OUT))
    return out[:N]
